# initial kernel scaffold (unmeasured)
import functools

import jax
import jax.numpy as jnp
from jax import lax
from jax.experimental import pallas as pl
from jax.experimental.pallas import tpu as pltpu

N_DEV = 4
M_BLK = 1024
K = 4096
N = 8192
TN = 2048
NT = N // TN


def kernel(x, w_mat):
    assert x.shape == (K, M_BLK * 1), (x.shape,)
    assert w_mat.shape == (K, N)

    def body(x_ref, w_ref, out_ref, g_ref, wbuf_ref, amax_ref,
             x_send, x_recv, a_send, a_recv, w_sem):
        me = lax.axis_index("i")

        barrier_sem = pltpu.get_barrier_semaphore()
        for d in range(1, N_DEV):
            pl.semaphore_signal(
                barrier_sem, inc=1,
                device_id=((me + d) % N_DEV,),
                device_id_type=pl.DeviceIdType.MESH,
            )
        pl.semaphore_wait(barrier_sem, N_DEV - 1)

        sends = []
        for d in range(1, N_DEV):
            dst = (me + d) % N_DEV
            rdma = pltpu.make_async_remote_copy(
                src_ref=x_ref.at[pl.ds(dst * M_BLK, M_BLK), :],
                dst_ref=g_ref.at[N_DEV - d],
                send_sem=x_send.at[d - 1],
                recv_sem=x_recv.at[N_DEV - d],
                device_id=(dst,),
                device_id_type=pl.DeviceIdType.MESH,
            )
            rdma.start()
            sends.append(rdma)

        amax = jnp.float32(0.0)
        offsets = [0, 1, 3, 2]
        for idx, doff in enumerate(offsets):
            s = (me + doff) % N_DEV
            if doff == 0:
                xblk = x_ref[pl.ds(me * M_BLK, M_BLK), :]
            else:
                recv = pltpu.make_async_remote_copy(
                    src_ref=g_ref.at[doff],
                    dst_ref=g_ref.at[doff],
                    send_sem=x_send.at[0],
                    recv_sem=x_recv.at[doff],
                    device_id=(s,),
                    device_id_type=pl.DeviceIdType.MESH,
                )
                recv.wait_recv()
                xblk = g_ref[doff]
            for nt in range(NT):
                cp = pltpu.make_async_copy(
                    w_ref.at[pl.ds(s * M_BLK, M_BLK), pl.ds(nt * TN, TN)],
                    wbuf_ref,
                    w_sem,
                )
                cp.start()
                cp.wait()
                part = jnp.dot(
                    xblk, wbuf_ref[...],
                    preferred_element_type=jnp.float32,
                )
                nsl = pl.ds(nt * TN, TN)
                if idx == 0:
                    out_ref[:, nsl] = part
                else:
                    tile = out_ref[:, nsl] + part
                    out_ref[:, nsl] = tile
                    if idx == len(offsets) - 1:
                        amax = jnp.maximum(amax, jnp.max(tile))

        amax_ref[0] = jnp.full((8, 128), amax, dtype=jnp.float32)
        a_sends = []
        for d in range(1, N_DEV):
            dst = (me + d) % N_DEV
            rdma = pltpu.make_async_remote_copy(
                src_ref=amax_ref.at[0],
                dst_ref=amax_ref.at[N_DEV - d],
                send_sem=a_send.at[d - 1],
                recv_sem=a_recv.at[N_DEV - d],
                device_id=(dst,),
                device_id_type=pl.DeviceIdType.MESH,
            )
            rdma.start()
            a_sends.append(rdma)
        for d in range(1, N_DEV):
            recv = pltpu.make_async_remote_copy(
                src_ref=amax_ref.at[d],
                dst_ref=amax_ref.at[d],
                send_sem=a_send.at[0],
                recv_sem=a_recv.at[d],
                device_id=((me + d) % N_DEV,),
                device_id_type=pl.DeviceIdType.MESH,
            )
            recv.wait_recv()
            amax = jnp.maximum(amax, amax_ref[d, 0, 0])

        scale = jnp.maximum(amax, jnp.float32(1e-30)) / 127.0
        for nt in range(NT):
            nsl = pl.ds(nt * TN, TN)
            y = jnp.maximum(out_ref[:, nsl], 0.0)
            q = jnp.clip(jnp.round(y / scale), -127.0, 127.0)
            out_ref[:, nsl] = q * scale

        for rdma in sends + a_sends:
            rdma.wait_send()

    return pl.pallas_call(
        body,
        out_shape=jax.ShapeDtypeStruct((M_BLK, N), jnp.float32),
        in_specs=[
            pl.BlockSpec(memory_space=pltpu.MemorySpace.VMEM),
            pl.BlockSpec(memory_space=pl.ANY),
        ],
        out_specs=pl.BlockSpec(memory_space=pltpu.MemorySpace.VMEM),
        scratch_shapes=[
            pltpu.VMEM((N_DEV, M_BLK, M_BLK), jnp.bfloat16),
            pltpu.VMEM((M_BLK, TN), jnp.bfloat16),
            pltpu.VMEM((N_DEV, 8, 128), jnp.float32),
            pltpu.SemaphoreType.DMA((N_DEV - 1,)),
            pltpu.SemaphoreType.DMA((N_DEV,)),
            pltpu.SemaphoreType.DMA((N_DEV - 1,)),
            pltpu.SemaphoreType.DMA((N_DEV,)),
            pltpu.SemaphoreType.DMA(()),
        ],
        compiler_params=pltpu.CompilerParams(collective_id=0),
    )(x, w_mat)


# baseline (device time: 175420 ns/iter reference)
import jax
import jax.numpy as jnp
from jax import lax
from jax.experimental import pallas as pl
from jax.experimental.pallas import tpu as pltpu

N_DEV = 4
M_BLK = 1024
K = 4096
N = 8192
TN = 1024
NT = N // TN
N_TILES = N_DEV * NT


def kernel(x, w_mat):
    assert x.shape == (K, M_BLK), (x.shape,)
    assert w_mat.shape == (K, N)
    x = x.astype(jnp.bfloat16)

    def body(x_ref, w_ref, out_ref, g_ref, wbuf_ref, amax_ref,
             x_send, x_recv, a_send, a_recv, w_sems):
        me = lax.axis_index("i")

        barrier_sem = pltpu.get_barrier_semaphore()
        for d in range(1, N_DEV):
            pl.semaphore_signal(
                barrier_sem, inc=1,
                device_id=((me + d) % N_DEV,),
                device_id_type=pl.DeviceIdType.MESH,
            )
        pl.semaphore_wait(barrier_sem, N_DEV - 1)

        sends = []
        for d in range(1, N_DEV):
            dst = (me + d) % N_DEV
            rdma = pltpu.make_async_remote_copy(
                src_ref=x_ref.at[pl.ds(dst * M_BLK, M_BLK), :],
                dst_ref=g_ref.at[3 - d],
                send_sem=x_send.at[d - 1],
                recv_sem=x_recv.at[3 - d],
                device_id=(dst,),
                device_id_type=pl.DeviceIdType.MESH,
            )
            rdma.start()
            sends.append(rdma)

        offsets = [0, 1, 3, 2]
        srcs = [(me + doff) % N_DEV for doff in offsets]

        def start_w(t, slot):
            j, nt = divmod(t, NT)
            cp = pltpu.make_async_copy(
                w_ref.at[pl.ds(srcs[j] * M_BLK, M_BLK),
                         pl.ds(nt * TN, TN)],
                wbuf_ref.at[slot],
                w_sems.at[slot],
            )
            cp.start()
            return cp

        amax = jnp.float32(0.0)
        xblk = None
        cps = [start_w(0, 0), None]
        for t in range(N_TILES):
            j, nt = divmod(t, NT)
            slot = t % 2
            if t + 1 < N_TILES:
                cps[1 - slot] = start_w(t + 1, 1 - slot)
            if nt == 0:
                doff = offsets[j]
                if doff == 0:
                    xblk = x_ref[pl.ds(me * M_BLK, M_BLK), :]
                else:
                    recv = pltpu.make_async_remote_copy(
                        src_ref=g_ref.at[doff - 1],
                        dst_ref=g_ref.at[doff - 1],
                        send_sem=x_send.at[0],
                        recv_sem=x_recv.at[doff - 1],
                        device_id=(srcs[j],),
                        device_id_type=pl.DeviceIdType.MESH,
                    )
                    recv.wait_recv()
                    xblk = g_ref[doff - 1]
            cps[slot].wait()
            part = jnp.dot(
                xblk, wbuf_ref[slot].astype(jnp.bfloat16),
                preferred_element_type=jnp.float32,
            )
            nsl = pl.ds(nt * TN, TN)
            if j == 0:
                out_ref[:, nsl] = part
            else:
                tile = out_ref[:, nsl] + part
                out_ref[:, nsl] = tile
                if j == N_DEV - 1:
                    amax = jnp.maximum(amax, jnp.max(tile))

        amax_ref[0] = jnp.full((8, 128), amax, dtype=jnp.float32)
        a_sends = []
        for d in range(1, N_DEV):
            dst = (me + d) % N_DEV
            rdma = pltpu.make_async_remote_copy(
                src_ref=amax_ref.at[0],
                dst_ref=amax_ref.at[N_DEV - d],
                send_sem=a_send.at[d - 1],
                recv_sem=a_recv.at[N_DEV - d],
                device_id=(dst,),
                device_id_type=pl.DeviceIdType.MESH,
            )
            rdma.start()
            a_sends.append(rdma)
        for d in range(1, N_DEV):
            recv = pltpu.make_async_remote_copy(
                src_ref=amax_ref.at[d],
                dst_ref=amax_ref.at[d],
                send_sem=a_send.at[0],
                recv_sem=a_recv.at[d],
                device_id=((me + d) % N_DEV,),
                device_id_type=pl.DeviceIdType.MESH,
            )
            recv.wait_recv()
            amax = jnp.maximum(amax, amax_ref[d, 0, 0])

        scale = jnp.maximum(amax, jnp.float32(1e-30)) / 127.0
        for nt in range(NT):
            nsl = pl.ds(nt * TN, TN)
            y = jnp.maximum(out_ref[:, nsl], 0.0)
            q = jnp.clip(jnp.round(y / scale), -127.0, 127.0)
            out_ref[:, nsl] = q * scale

        for rdma in sends + a_sends:
            rdma.wait_send()

    return pl.pallas_call(
        body,
        out_shape=jax.ShapeDtypeStruct((M_BLK, N), jnp.float32),
        in_specs=[
            pl.BlockSpec(memory_space=pltpu.MemorySpace.VMEM),
            pl.BlockSpec(memory_space=pl.ANY),
        ],
        out_specs=pl.BlockSpec(memory_space=pltpu.MemorySpace.VMEM),
        scratch_shapes=[
            pltpu.VMEM((N_DEV - 1, M_BLK, M_BLK), jnp.bfloat16),
            pltpu.VMEM((2, M_BLK, TN), jnp.float32),
            pltpu.VMEM((N_DEV, 8, 128), jnp.float32),
            pltpu.SemaphoreType.DMA((N_DEV - 1,)),
            pltpu.SemaphoreType.DMA((N_DEV - 1,)),
            pltpu.SemaphoreType.DMA((N_DEV - 1,)),
            pltpu.SemaphoreType.DMA((N_DEV,)),
            pltpu.SemaphoreType.DMA((2,)),
        ],
        compiler_params=pltpu.CompilerParams(
            collective_id=0,
            vmem_limit_bytes=110 * 1024 * 1024,
        ),
    )(x, w_mat)


# device time: 121273 ns/iter; 1.4465x vs baseline; 1.4465x over previous
import os

import jax
import jax.numpy as jnp
from jax import lax
from jax.experimental import pallas as pl
from jax.experimental.pallas import tpu as pltpu

_SKIP_COMM = os.environ.get("KERNEL_SKIP_COMM") == "1"
_SKIP_W = os.environ.get("KERNEL_SKIP_W") == "1"

N_DEV = 4
M_BLK = 1024
K = 4096
N = 8192
TN = 1024
NT = N // TN
N_TILES = N_DEV * NT


def kernel(x, w_mat):
    assert x.shape == (K, M_BLK), (x.shape,)
    assert w_mat.shape == (K, N)
    x = x.astype(jnp.bfloat16)

    def body(x_ref, w_ref, out_ref, g_ref, wbuf_ref, amax_ref,
             x_send, x_recv, a_send, a_recv, w_sems):
        me = lax.axis_index("i")

        sends = []
        if not _SKIP_COMM:
            barrier_sem = pltpu.get_barrier_semaphore()
            for d in range(1, N_DEV):
                pl.semaphore_signal(
                    barrier_sem, inc=1,
                    device_id=((me + d) % N_DEV,),
                    device_id_type=pl.DeviceIdType.MESH,
                )
            pl.semaphore_wait(barrier_sem, N_DEV - 1)

            for d in range(1, N_DEV):
                dst = (me + d) % N_DEV
                rdma = pltpu.make_async_remote_copy(
                    src_ref=x_ref.at[pl.ds(dst * M_BLK, M_BLK), :],
                    dst_ref=g_ref.at[3 - d],
                    send_sem=x_send.at[d - 1],
                    recv_sem=x_recv.at[3 - d],
                    device_id=(dst,),
                    device_id_type=pl.DeviceIdType.MESH,
                )
                rdma.start()
                sends.append(rdma)

        offsets = [0, 1, 3, 2]
        srcs = [(me + doff) % N_DEV for doff in offsets]

        def start_w(t, slot):
            j, nt = divmod(t, NT)
            cp = pltpu.make_async_copy(
                w_ref.at[pl.ds(srcs[j] * M_BLK, M_BLK),
                         pl.ds(nt * TN, TN)],
                wbuf_ref.at[slot],
                w_sems.at[slot],
            )
            cp.start()
            return cp

        amax = jnp.float32(0.0)
        xblk = None
        cps = [start_w(0, 0), None]
        if _SKIP_W:
            cps[0].wait()
        for t in range(N_TILES):
            j, nt = divmod(t, NT)
            slot = 0 if _SKIP_W else t % 2
            if not _SKIP_W and t + 1 < N_TILES:
                cps[1 - slot] = start_w(t + 1, 1 - slot)
            if nt == 0:
                doff = offsets[j]
                if doff == 0 or _SKIP_COMM:
                    xblk = x_ref[pl.ds(me * M_BLK, M_BLK), :]
                else:
                    recv = pltpu.make_async_remote_copy(
                        src_ref=g_ref.at[doff - 1],
                        dst_ref=g_ref.at[doff - 1],
                        send_sem=x_send.at[0],
                        recv_sem=x_recv.at[doff - 1],
                        device_id=(srcs[j],),
                        device_id_type=pl.DeviceIdType.MESH,
                    )
                    recv.wait_recv()
                    xblk = g_ref[doff - 1]
            if not _SKIP_W:
                cps[slot].wait()
            part = jnp.dot(
                xblk, wbuf_ref[slot].astype(jnp.bfloat16),
                preferred_element_type=jnp.float32,
            )
            nsl = pl.ds(nt * TN, TN)
            if j == 0:
                out_ref[:, nsl] = part
            else:
                tile = out_ref[:, nsl] + part
                out_ref[:, nsl] = tile
                if j == N_DEV - 1:
                    amax = jnp.maximum(amax, jnp.max(tile))

        amax_ref[0] = jnp.full((8, 128), amax, dtype=jnp.float32)
        a_sends = []
        for d in range(1, N_DEV) if not _SKIP_COMM else []:
            dst = (me + d) % N_DEV
            rdma = pltpu.make_async_remote_copy(
                src_ref=amax_ref.at[0],
                dst_ref=amax_ref.at[N_DEV - d],
                send_sem=a_send.at[d - 1],
                recv_sem=a_recv.at[N_DEV - d],
                device_id=(dst,),
                device_id_type=pl.DeviceIdType.MESH,
            )
            rdma.start()
            a_sends.append(rdma)
        for d in range(1, N_DEV) if not _SKIP_COMM else []:
            recv = pltpu.make_async_remote_copy(
                src_ref=amax_ref.at[d],
                dst_ref=amax_ref.at[d],
                send_sem=a_send.at[0],
                recv_sem=a_recv.at[d],
                device_id=((me + d) % N_DEV,),
                device_id_type=pl.DeviceIdType.MESH,
            )
            recv.wait_recv()
            amax = jnp.maximum(amax, amax_ref[d, 0, 0])

        scale = jnp.maximum(amax, jnp.float32(1e-30)) / 127.0
        for nt in range(NT):
            nsl = pl.ds(nt * TN, TN)
            y = jnp.maximum(out_ref[:, nsl], 0.0)
            q = jnp.clip(jnp.round(y / scale), -127.0, 127.0)
            out_ref[:, nsl] = q * scale

        for rdma in sends + a_sends:
            rdma.wait_send()

    return pl.pallas_call(
        body,
        out_shape=jax.ShapeDtypeStruct((M_BLK, N), jnp.float32),
        in_specs=[
            pl.BlockSpec(memory_space=pltpu.MemorySpace.VMEM),
            pl.BlockSpec(memory_space=pl.ANY),
        ],
        out_specs=pl.BlockSpec(memory_space=pltpu.MemorySpace.VMEM),
        scratch_shapes=[
            pltpu.VMEM((N_DEV - 1, M_BLK, M_BLK), jnp.bfloat16),
            pltpu.VMEM((2, M_BLK, TN), jnp.float32),
            pltpu.VMEM((N_DEV, 8, 128), jnp.float32),
            pltpu.SemaphoreType.DMA((N_DEV - 1,)),
            pltpu.SemaphoreType.DMA((N_DEV - 1,)),
            pltpu.SemaphoreType.DMA((N_DEV - 1,)),
            pltpu.SemaphoreType.DMA((N_DEV,)),
            pltpu.SemaphoreType.DMA((2,)),
        ],
        compiler_params=pltpu.CompilerParams(
            collective_id=None if _SKIP_COMM else 0,
            vmem_limit_bytes=110 * 1024 * 1024,
        ),
    )(x, w_mat)
